# baseline (device time: 19847 ns/iter reference)
import jax
import jax.numpy as jnp
from jax import lax
from jax.experimental import pallas as pl
from jax.experimental.pallas import tpu as pltpu

N_DEV = 8
M_PER = 128
M = 1024
N_COLS = 1024

_GROUPS = ((0, 384), (384, 384), (768, 256))


def _perm(order):
    out = []
    for np_ in range(8):
        bits = {a: (np_ >> (2 - i)) & 1 for i, a in enumerate(order)}
        x, y, z = bits["x"], bits["y"], bits["z"]
        out.append(4 * z + 2 * y + (x ^ y))
    return tuple(out)


_ORDERS = (("z", "y", "x"), ("y", "x", "z"), ("x", "z", "y"))
_PERMS = tuple(_perm(o) for o in _ORDERS)


def kernel(x, w_mat):
    bf = jnp.bfloat16
    f32 = jnp.float32

    def body(x_ref, w_ref, out_ref,
             p0, p1, p2, xg0, xg1, xg2, wb,
             r00, r01, r02, r10, r11, r12, r20, r21, r22,
             send_sems, recv_sems):
        p_refs = (p0, p1, p2)
        xg_refs = (xg0, xg1, xg2)
        rv_refs = ((r00, r01, r02), (r10, r11, r12), (r20, r21, r22))

        my = lax.axis_index("i")
        q = my % 4
        my_z = my // 4
        my_y = jnp.where(q >= 2, 1, 0)
        my_x = jnp.where((q == 1) | (q == 2), 1, 0)
        pz = my ^ 4
        py = my - q + (3 - q)
        px = my - q + (q ^ 1)

        coord = {"x": my_x, "y": my_y, "z": my_z}
        partner = {"x": px, "y": py, "z": pz}

        barrier_sem = pltpu.get_barrier_semaphore()
        for nbr in (pz, py, px):
            pl.semaphore_signal(
                barrier_sem, inc=1,
                device_id=(nbr,), device_id_type=pl.DeviceIdType.MESH,
            )
        pl.semaphore_wait(barrier_sem, 3)

        wb[...] = w_ref[...].astype(bf)
        for g in range(3):
            for np_, cid in enumerate(_PERMS[g]):
                xg_refs[g][np_ * M_PER:(np_ + 1) * M_PER, :] = (
                    x_ref[cid * M_PER:(cid + 1) * M_PER, :].astype(bf)
                )

        def start_rdma(g, p, row0, nrows, axis):
            rdma = pltpu.make_async_remote_copy(
                src_ref=p_refs[g].at[pl.ds(row0, nrows), :],
                dst_ref=rv_refs[g][p],
                send_sem=send_sems.at[g, p],
                recv_sem=recv_sems.at[g, p],
                device_id=(partner[axis],),
                device_id_type=pl.DeviceIdType.MESH,
            )
            rdma.start()
            return rdma

        inflight = []
        keep0s = []
        for g, (c0, nc) in enumerate(_GROUPS):
            cA = coord[_ORDERS[g][0]]
            s0 = (1 - cA) * 512
            k0 = cA * 512
            p_refs[g][pl.ds(s0, 512), :] = jnp.dot(
                xg_refs[g][pl.ds(s0, 512), :], wb[:, c0:c0 + nc],
                preferred_element_type=f32,
            ).astype(bf)
            inflight.append(start_rdma(g, 0, s0, 512, _ORDERS[g][0]))
            keep0s.append(k0)
        for g, (c0, nc) in enumerate(_GROUPS):
            p_refs[g][pl.ds(keep0s[g], 512), :] = jnp.dot(
                xg_refs[g][pl.ds(keep0s[g], 512), :], wb[:, c0:c0 + nc],
                preferred_element_type=f32,
            ).astype(bf)

        def acc(g, row0, rv_ref, rel0, nrows):
            p_refs[g][pl.ds(row0, nrows), :] = (
                p_refs[g][pl.ds(row0, nrows), :].astype(f32)
                + rv_ref[pl.ds(rel0, nrows), :].astype(f32)
            ).astype(bf)

        for p in (1, 2):
            for g in range(3):
                order = _ORDERS[g]
                nrows = 512 >> p
                prefix = 0
                for j in range(p):
                    prefix = prefix + coord[order[j]] * (512 >> j)
                c_p = coord[order[p]]
                send_row0 = prefix + (1 - c_p) * nrows
                keep_row0 = prefix + c_p * nrows
                inflight[g].wait()
                acc(g, send_row0, rv_refs[g][p - 1], (1 - c_p) * nrows,
                    nrows)
                inflight[g] = start_rdma(g, p, send_row0, nrows, order[p])
                acc(g, keep_row0, rv_refs[g][p - 1], c_p * nrows, nrows)

        for g, (c0, nc) in enumerate(_GROUPS):
            order = _ORDERS[g]
            fin = (coord[order[0]] * 512 + coord[order[1]] * 256
                   + coord[order[2]] * 128)
            inflight[g].wait()
            out_ref[:, c0:c0 + nc] = (
                p_refs[g][pl.ds(fin, M_PER), :].astype(f32)
                + rv_refs[g][2][...].astype(f32)
            )

    scratch = [
        pltpu.VMEM((M, 384), bf),
        pltpu.VMEM((M, 384), bf),
        pltpu.VMEM((M, 256), bf),
        pltpu.VMEM((M, 128), bf),
        pltpu.VMEM((M, 128), bf),
        pltpu.VMEM((M, 128), bf),
        pltpu.VMEM((128, N_COLS), bf),
        pltpu.VMEM((512, 384), bf),
        pltpu.VMEM((256, 384), bf),
        pltpu.VMEM((128, 384), bf),
        pltpu.VMEM((512, 384), bf),
        pltpu.VMEM((256, 384), bf),
        pltpu.VMEM((128, 384), bf),
        pltpu.VMEM((512, 256), bf),
        pltpu.VMEM((256, 256), bf),
        pltpu.VMEM((128, 256), bf),
        pltpu.SemaphoreType.DMA((3, 3)),
        pltpu.SemaphoreType.DMA((3, 3)),
    ]
    return pl.pallas_call(
        body,
        out_shape=jax.ShapeDtypeStruct((M_PER, N_COLS), jnp.float32),
        in_specs=[
            pl.BlockSpec(memory_space=pltpu.VMEM),
            pl.BlockSpec(memory_space=pltpu.VMEM),
        ],
        out_specs=pl.BlockSpec(memory_space=pltpu.VMEM),
        scratch_shapes=scratch,
        compiler_params=pltpu.CompilerParams(collective_id=0),
    )(x, w_mat)


# device time: 17578 ns/iter; 1.1291x vs baseline; 1.1291x over previous
import jax
import jax.numpy as jnp
from jax import lax
from jax.experimental import pallas as pl
from jax.experimental.pallas import tpu as pltpu

N_DEV = 8
M_PER = 128
M = 1024
N_COLS = 1024

_GROUPS = ((0, 384), (384, 384), (768, 256))


def _perm(order):
    out = []
    for np_ in range(8):
        bits = {a: (np_ >> (2 - i)) & 1 for i, a in enumerate(order)}
        x, y, z = bits["x"], bits["y"], bits["z"]
        out.append(4 * z + 2 * y + (x ^ y))
    return tuple(out)


_ORDERS = (("z", "y", "x"), ("y", "x", "z"), ("x", "z", "y"))
_PERMS = tuple(_perm(o) for o in _ORDERS)


def kernel(x, w_mat):
    bf = jnp.bfloat16
    f32 = jnp.float32

    def body(x_ref, w_ref, out_ref,
             p0, p1, p2, xg0, xg1, xg2, wb,
             r00, r01, r02, r10, r11, r12, r20, r21, r22,
             send_sems, recv_sems):
        p_refs = (p0, p1, p2)
        xg_refs = (xg0, xg1, xg2)
        rv_refs = ((r00, r01, r02), (r10, r11, r12), (r20, r21, r22))

        my = lax.axis_index("i")
        q = my % 4
        my_z = my // 4
        my_y = jnp.where(q >= 2, 1, 0)
        my_x = jnp.where((q == 1) | (q == 2), 1, 0)
        pz = my ^ 4
        py = my - q + (3 - q)
        px = my - q + (q ^ 1)

        coord = {"x": my_x, "y": my_y, "z": my_z}
        partner = {"x": px, "y": py, "z": pz}

        barrier_sem = pltpu.get_barrier_semaphore()
        for nbr in (pz, py, px):
            pl.semaphore_signal(
                barrier_sem, inc=1,
                device_id=(nbr,), device_id_type=pl.DeviceIdType.MESH,
            )

        wb[...] = w_ref[...].astype(bf)
        for g in range(3):
            for np_, cid in enumerate(_PERMS[g]):
                xg_refs[g][np_ * M_PER:(np_ + 1) * M_PER, :] = (
                    x_ref[cid * M_PER:(cid + 1) * M_PER, :].astype(bf)
                )

        pl.semaphore_wait(barrier_sem, 3)

        cs = [[coord[a] for a in _ORDERS[g]] for g in range(3)]
        def start_rdma(g, sem, src_row0, rv_ref, rel0, nrows, axis):
            rdma = pltpu.make_async_remote_copy(
                src_ref=p_refs[g].at[pl.ds(src_row0, nrows), :],
                dst_ref=rv_ref.at[pl.ds(rel0, nrows), :],
                send_sem=send_sems.at[g, sem],
                recv_sem=recv_sems.at[g, sem],
                device_id=(partner[axis],),
                device_id_type=pl.DeviceIdType.MESH,
            )
            rdma.start()
            return rdma

        def acc(g, row0, rv_ref, rel0, nrows):
            p_refs[g][pl.ds(row0, nrows), :] = (
                p_refs[g][pl.ds(row0, nrows), :].astype(f32)
                + rv_ref[pl.ds(rel0, nrows), :].astype(f32)
            ).astype(bf)

        k0 = [cs[g][0] * 512 for g in range(3)]
        s0 = [(1 - cs[g][0]) * 512 for g in range(3)]
        p1_send = [k0[g] + (1 - cs[g][1]) * 256 for g in range(3)]
        prefix2 = [k0[g] + cs[g][1] * 256 for g in range(3)]
        p2_send = [prefix2[g] + (1 - cs[g][2]) * 128 for g in range(3)]
        fin = [prefix2[g] + cs[g][2] * 128 for g in range(3)]

        p0A, p0B = [None] * 3, [None] * 3
        for g, (c0, nc) in enumerate(_GROUPS):
            p_refs[g][pl.ds(s0[g], 512), :] = jnp.dot(
                xg_refs[g][pl.ds(s0[g], 512), :], wb[:, c0:c0 + nc],
                preferred_element_type=f32,
            ).astype(bf)
            relA = (1 - cs[g][1]) * 256
            relB = cs[g][1] * 256
            p0A[g] = start_rdma(g, 0, s0[g] + relA, rv_refs[g][0], relA,
                                256, _ORDERS[g][0])
            p0B[g] = start_rdma(g, 1, s0[g] + relB, rv_refs[g][0], relB,
                                256, _ORDERS[g][0])
        for g, (c0, nc) in enumerate(_GROUPS):
            p_refs[g][pl.ds(k0[g], 512), :] = jnp.dot(
                xg_refs[g][pl.ds(k0[g], 512), :], wb[:, c0:c0 + nc],
                preferred_element_type=f32,
            ).astype(bf)

        p1A, p1B = [None] * 3, [None] * 3
        for g in range(3):
            p0A[g].wait()
            acc(g, p1_send[g], rv_refs[g][0], (1 - cs[g][1]) * 256, 256)
            relA = (1 - cs[g][2]) * 128
            relB = cs[g][2] * 128
            p1A[g] = start_rdma(g, 2, p1_send[g] + relA, rv_refs[g][1],
                                relA, 128, _ORDERS[g][1])
            p1B[g] = start_rdma(g, 3, p1_send[g] + relB, rv_refs[g][1],
                                relB, 128, _ORDERS[g][1])
        for g in range(3):
            p0B[g].wait()
            acc(g, prefix2[g], rv_refs[g][0], cs[g][1] * 256, 256)

        p2r = [None] * 3
        for g in range(3):
            p1A[g].wait()
            acc(g, p2_send[g], rv_refs[g][1], (1 - cs[g][2]) * 128, 128)
            p2r[g] = start_rdma(g, 4, p2_send[g], rv_refs[g][2], 0, 128,
                                _ORDERS[g][2])
        for g in range(3):
            p1B[g].wait()
            acc(g, fin[g], rv_refs[g][1], cs[g][2] * 128, 128)

        for g, (c0, nc) in enumerate(_GROUPS):
            p2r[g].wait()
            out_ref[:, c0:c0 + nc] = (
                p_refs[g][pl.ds(fin[g], M_PER), :].astype(f32)
                + rv_refs[g][2][...].astype(f32)
            )

    scratch = [
        pltpu.VMEM((M, 384), bf),
        pltpu.VMEM((M, 384), bf),
        pltpu.VMEM((M, 256), bf),
        pltpu.VMEM((M, 128), bf),
        pltpu.VMEM((M, 128), bf),
        pltpu.VMEM((M, 128), bf),
        pltpu.VMEM((128, N_COLS), bf),
        pltpu.VMEM((512, 384), bf),
        pltpu.VMEM((256, 384), bf),
        pltpu.VMEM((128, 384), bf),
        pltpu.VMEM((512, 384), bf),
        pltpu.VMEM((256, 384), bf),
        pltpu.VMEM((128, 384), bf),
        pltpu.VMEM((512, 256), bf),
        pltpu.VMEM((256, 256), bf),
        pltpu.VMEM((128, 256), bf),
        pltpu.SemaphoreType.DMA((3, 5)),
        pltpu.SemaphoreType.DMA((3, 5)),
    ]
    return pl.pallas_call(
        body,
        out_shape=jax.ShapeDtypeStruct((M_PER, N_COLS), jnp.float32),
        in_specs=[
            pl.BlockSpec(memory_space=pltpu.VMEM),
            pl.BlockSpec(memory_space=pltpu.VMEM),
        ],
        out_specs=pl.BlockSpec(memory_space=pltpu.VMEM),
        scratch_shapes=scratch,
        compiler_params=pltpu.CompilerParams(collective_id=0),
    )(x, w_mat)


# device time: 16941 ns/iter; 1.1715x vs baseline; 1.0376x over previous
import jax
import jax.numpy as jnp
from jax import lax
from jax.experimental import pallas as pl
from jax.experimental.pallas import tpu as pltpu

N_DEV = 8
M_PER = 128
M = 1024
N_COLS = 1024

_GROUPS = ((0, 384), (384, 384), (768, 256))


def _perm(order):
    out = []
    for np_ in range(8):
        bits = {a: (np_ >> (2 - i)) & 1 for i, a in enumerate(order)}
        x, y, z = bits["x"], bits["y"], bits["z"]
        out.append(4 * z + 2 * y + (x ^ y))
    return tuple(out)


_ORDERS = (("z", "y", "x"), ("y", "x", "z"), ("x", "z", "y"))
_PERMS = tuple(_perm(o) for o in _ORDERS)


def kernel(x, w_mat):
    bf = jnp.bfloat16
    f32 = jnp.float32

    def body(x_ref, w_ref, out_ref,
             p0, p1, p2, xg0, xg1, xg2, wb,
             r00, r01, r02, r10, r11, r12, r20, r21, r22,
             send_sems, recv_sems):
        p_refs = (p0, p1, p2)
        xg_refs = (xg0, xg1, xg2)
        rv_refs = ((r00, r01, r02), (r10, r11, r12), (r20, r21, r22))

        my = lax.axis_index("i")
        q = my % 4
        my_z = my // 4
        my_y = jnp.where(q >= 2, 1, 0)
        my_x = jnp.where((q == 1) | (q == 2), 1, 0)
        pz = my ^ 4
        py = my - q + (3 - q)
        px = my - q + (q ^ 1)

        coord = {"x": my_x, "y": my_y, "z": my_z}
        partner = {"x": px, "y": py, "z": pz}

        barrier_sem = pltpu.get_barrier_semaphore()
        for nbr in (pz, py, px):
            pl.semaphore_signal(
                barrier_sem, inc=1,
                device_id=(nbr,), device_id_type=pl.DeviceIdType.MESH,
            )

        wb[...] = w_ref[...].astype(bf)
        for g in range(3):
            for np_, cid in enumerate(_PERMS[g]):
                xg_refs[g][np_ * M_PER:(np_ + 1) * M_PER, :] = (
                    x_ref[cid * M_PER:(cid + 1) * M_PER, :].astype(bf)
                )

        cs = [[coord[a] for a in _ORDERS[g]] for g in range(3)]
        def start_rdma(g, sem, src_row0, rv_ref, rel0, nrows, axis):
            rdma = pltpu.make_async_remote_copy(
                src_ref=p_refs[g].at[pl.ds(src_row0, nrows), :],
                dst_ref=rv_ref.at[pl.ds(rel0, nrows), :],
                send_sem=send_sems.at[g, sem],
                recv_sem=recv_sems.at[g, sem],
                device_id=(partner[axis],),
                device_id_type=pl.DeviceIdType.MESH,
            )
            rdma.start()
            return rdma

        def acc(g, row0, rv_ref, rel0, nrows):
            p_refs[g][pl.ds(row0, nrows), :] = (
                p_refs[g][pl.ds(row0, nrows), :].astype(f32)
                + rv_ref[pl.ds(rel0, nrows), :].astype(f32)
            ).astype(bf)

        k0 = [cs[g][0] * 512 for g in range(3)]
        s0 = [(1 - cs[g][0]) * 512 for g in range(3)]
        p1_send = [k0[g] + (1 - cs[g][1]) * 256 for g in range(3)]
        prefix2 = [k0[g] + cs[g][1] * 256 for g in range(3)]
        p2_send = [prefix2[g] + (1 - cs[g][2]) * 128 for g in range(3)]
        fin = [prefix2[g] + cs[g][2] * 128 for g in range(3)]

        for g, (c0, nc) in enumerate(_GROUPS):
            p_refs[g][pl.ds(s0[g], 512), :] = jnp.dot(
                xg_refs[g][pl.ds(s0[g], 512), :], wb[:, c0:c0 + nc],
                preferred_element_type=f32,
            ).astype(bf)
        pl.semaphore_wait(barrier_sem, 3)
        p0A, p0B = [None] * 3, [None] * 3
        for g in range(3):
            relA = (1 - cs[g][1]) * 256
            relB = cs[g][1] * 256
            p0A[g] = start_rdma(g, 0, s0[g] + relA, rv_refs[g][0], relA,
                                256, _ORDERS[g][0])
            p0B[g] = start_rdma(g, 1, s0[g] + relB, rv_refs[g][0], relB,
                                256, _ORDERS[g][0])
        for g, (c0, nc) in enumerate(_GROUPS):
            p_refs[g][pl.ds(k0[g], 512), :] = jnp.dot(
                xg_refs[g][pl.ds(k0[g], 512), :], wb[:, c0:c0 + nc],
                preferred_element_type=f32,
            ).astype(bf)

        p1A, p1B = [None] * 3, [None] * 3
        for g in range(3):
            p0A[g].wait()
            acc(g, p1_send[g], rv_refs[g][0], (1 - cs[g][1]) * 256, 256)
            relA = (1 - cs[g][2]) * 128
            relB = cs[g][2] * 128
            p1A[g] = start_rdma(g, 2, p1_send[g] + relA, rv_refs[g][1],
                                relA, 128, _ORDERS[g][1])
            p1B[g] = start_rdma(g, 3, p1_send[g] + relB, rv_refs[g][1],
                                relB, 128, _ORDERS[g][1])
        for g in range(3):
            p0B[g].wait()
            acc(g, prefix2[g], rv_refs[g][0], cs[g][1] * 256, 256)

        p2r = [None] * 3
        for g in range(3):
            p1A[g].wait()
            acc(g, p2_send[g], rv_refs[g][1], (1 - cs[g][2]) * 128, 128)
            p2r[g] = start_rdma(g, 4, p2_send[g], rv_refs[g][2], 0, 128,
                                _ORDERS[g][2])
        for g in range(3):
            p1B[g].wait()
            acc(g, fin[g], rv_refs[g][1], cs[g][2] * 128, 128)

        for g, (c0, nc) in enumerate(_GROUPS):
            p2r[g].wait()
            out_ref[:, c0:c0 + nc] = (
                p_refs[g][pl.ds(fin[g], M_PER), :].astype(f32)
                + rv_refs[g][2][...].astype(f32)
            )

    scratch = [
        pltpu.VMEM((M, 384), bf),
        pltpu.VMEM((M, 384), bf),
        pltpu.VMEM((M, 256), bf),
        pltpu.VMEM((M, 128), bf),
        pltpu.VMEM((M, 128), bf),
        pltpu.VMEM((M, 128), bf),
        pltpu.VMEM((128, N_COLS), bf),
        pltpu.VMEM((512, 384), bf),
        pltpu.VMEM((256, 384), bf),
        pltpu.VMEM((128, 384), bf),
        pltpu.VMEM((512, 384), bf),
        pltpu.VMEM((256, 384), bf),
        pltpu.VMEM((128, 384), bf),
        pltpu.VMEM((512, 256), bf),
        pltpu.VMEM((256, 256), bf),
        pltpu.VMEM((128, 256), bf),
        pltpu.SemaphoreType.DMA((3, 5)),
        pltpu.SemaphoreType.DMA((3, 5)),
    ]
    return pl.pallas_call(
        body,
        out_shape=jax.ShapeDtypeStruct((M_PER, N_COLS), jnp.float32),
        in_specs=[
            pl.BlockSpec(memory_space=pltpu.VMEM),
            pl.BlockSpec(memory_space=pltpu.VMEM),
        ],
        out_specs=pl.BlockSpec(memory_space=pltpu.VMEM),
        scratch_shapes=scratch,
        compiler_params=pltpu.CompilerParams(collective_id=0),
    )(x, w_mat)
